# Initial kernel scaffold; baseline (speedup 1.0000x reference)
import functools

import jax
import jax.numpy as jnp
from jax import lax
from jax.experimental import pallas as pl
from jax.experimental.pallas import tpu as pltpu
from jax.experimental.pallas import tpu_sc as plsc

EPS = 0.001
H = 0.1
NC, NS = 2, 16
NW = NC * NS
CH = 400
EDGE_BLOCK = 4000


def _tv_tanh(t):
    t = t - jnp.mean(t, axis=1, keepdims=True)
    t = t * lax.rsqrt(jnp.sum(t * t, axis=1, keepdims=True) + EPS)
    return jnp.tanh(t)


def _dot(a, b):
    return jnp.dot(a, b, preferred_element_type=jnp.float32)


def _sc_mesh():
    return plsc.VectorSubcoreMesh(
        core_axis_name="c", subcore_axis_name="s", num_cores=NC,
        num_subcores=NS)


def _sc_gather(z1, z2, i_ind, j_ind):
    e = i_ind.shape[0]
    per = e // NW
    steps = per // CH

    @functools.partial(
        pl.kernel,
        out_type=(jax.ShapeDtypeStruct((e, 64), jnp.float32),
                  jax.ShapeDtypeStruct((e, 64), jnp.float32)),
        mesh=_sc_mesh(),
        scratch_types=[
            pltpu.VMEM((CH,), jnp.int32),
            pltpu.VMEM((CH,), jnp.int32),
            pltpu.VMEM((CH, 64), jnp.float32),
            pltpu.VMEM((CH, 64), jnp.float32),
            pltpu.SemaphoreType.DMA,
            pltpu.SemaphoreType.DMA,
        ],
    )
    def k(z1_hbm, z2_hbm, i_hbm, j_hbm, g1_hbm, g2_hbm,
          idx1_v, idx2_v, buf1_v, buf2_v, sem1, sem2):
        wid = lax.axis_index("s") * NC + lax.axis_index("c")
        base = wid * per

        def body(t, carry):
            start = base + t * CH
            pltpu.sync_copy(i_hbm.at[pl.ds(start, CH)], idx1_v)
            pltpu.sync_copy(j_hbm.at[pl.ds(start, CH)], idx2_v)
            d1 = pltpu.async_copy(z1_hbm.at[idx1_v], buf1_v, sem1)
            d2 = pltpu.async_copy(z2_hbm.at[idx2_v], buf2_v, sem2)
            d1.wait()
            d2.wait()
            pltpu.sync_copy(buf1_v, g1_hbm.at[pl.ds(start, CH)])
            pltpu.sync_copy(buf2_v, g2_hbm.at[pl.ds(start, CH)])
            return carry

        lax.fori_loop(0, steps, body, 0)

    return k(z1, z2, i_ind, j_ind)


def _sc_scatter(flux, i_ind, j_ind, zeros_n):
    e = i_ind.shape[0]
    n = zeros_n.shape[0]
    per = e // NW
    steps = per // CH
    stripe = n // NS

    @functools.partial(
        pl.kernel,
        out_type=jax.ShapeDtypeStruct((NC, 2 * n, 64), jnp.float32),
        mesh=_sc_mesh(),
        scratch_types=[
            pltpu.VMEM((CH,), jnp.int32),
            pltpu.VMEM((CH,), jnp.int32),
            pltpu.VMEM((CH, 64), jnp.float32),
            pltpu.VMEM_SHARED((n, 64), jnp.float32),
            pltpu.VMEM_SHARED((n, 64), jnp.float32),
        ],
    )
    def k(flux_hbm, i_hbm, j_hbm, z_hbm, out_hbm,
          idx1_v, idx2_v, buf_v, acc1, acc2):
        cid = lax.axis_index("c")
        sid = lax.axis_index("s")
        wid = sid * NC + cid
        row0 = sid * stripe
        pltpu.sync_copy(z_hbm.at[pl.ds(row0, stripe)],
                        acc1.at[pl.ds(row0, stripe)])
        pltpu.sync_copy(z_hbm.at[pl.ds(row0, stripe)],
                        acc2.at[pl.ds(row0, stripe)])
        plsc.subcore_barrier()

        base = wid * per

        def body(t, carry):
            start = base + t * CH
            pltpu.sync_copy(i_hbm.at[pl.ds(start, CH)], idx1_v)
            pltpu.sync_copy(j_hbm.at[pl.ds(start, CH)], idx2_v)
            pltpu.sync_copy(flux_hbm.at[pl.ds(start, CH)], buf_v)
            pltpu.sync_copy(buf_v, acc1.at[idx1_v], add=True)
            pltpu.sync_copy(buf_v, acc2.at[idx2_v], add=True)
            return carry

        lax.fori_loop(0, steps, body, 0)
        plsc.subcore_barrier()
        pltpu.sync_copy(acc1.at[pl.ds(row0, stripe)],
                        out_hbm.at[cid, pl.ds(row0, stripe)])
        pltpu.sync_copy(acc2.at[pl.ds(row0, stripe)],
                        out_hbm.at[cid, pl.ds(n + row0, stripe)])

    return k(flux, i_ind, j_ind, zeros_n)


def _w_spec():
    return pl.BlockSpec((64, 64), lambda i: (0, 0))


def _node_open_body(x_ref, w1, w2, b1, b2, xn_out, z1_out, z2_out):
    t = _dot(x_ref[...], w1[...])
    t = _tv_tanh(t)
    xn = _dot(t, w2[...])
    xn_out[...] = xn
    z1_out[...] = _dot(xn, b1[...])
    z2_out[...] = _dot(xn, b2[...])


def _node_open(xn0, w1t, w2t, b1, b2):
    n = xn0.shape[0]
    shp = jax.ShapeDtypeStruct((n, 64), jnp.float32)
    return pl.pallas_call(
        _node_open_body,
        grid=(1,),
        in_specs=[
            pl.BlockSpec((n, 128), lambda i: (0, 0)),
            pl.BlockSpec((128, 64), lambda i: (0, 0)),
            _w_spec(), _w_spec(), _w_spec(),
        ],
        out_specs=[pl.BlockSpec((n, 64), lambda i: (0, 0))] * 3,
        out_shape=(shp, shp, shp),
    )(xn0, w1t, w2t, b1, b2)


def _edge0_body(xe_ref, g1_ref, g2_ref, w1, w2, a2, k2, xe_out, dxe_out):
    t = _dot(xe_ref[...], w1[...])
    t = _tv_tanh(t)
    xe = _dot(t, w2[...])
    u = g1_ref[...] + g2_ref[...] + _dot(xe, a2[...])
    u = _tv_tanh(u)
    d = _dot(u, k2[...])
    dxe_out[...] = d
    xe_out[...] = xe - H * d


def _edge0(xe16, g1, g2, w1t, w2t, a2t, ke2t):
    e = xe16.shape[0]
    grid = (e // EDGE_BLOCK,)
    eb = pl.BlockSpec((EDGE_BLOCK, 64), lambda i: (i, 0))
    shp = jax.ShapeDtypeStruct((e, 64), jnp.float32)
    return pl.pallas_call(
        _edge0_body,
        grid=grid,
        in_specs=[
            pl.BlockSpec((EDGE_BLOCK, 16), lambda i: (i, 0)),
            eb, eb,
            pl.BlockSpec((16, 64), lambda i: (0, 0)),
            _w_spec(), _w_spec(), _w_spec(),
        ],
        out_specs=[eb, eb],
        out_shape=(shp, shp),
    )(xe16, g1, g2, w1t, w2t, a2t, ke2t)


def _edge1_body(xe_ref, g1_ref, g2_ref, a2, k2, xe_out, dxe_out):
    xe = xe_ref[...]
    u = g1_ref[...] + g2_ref[...] + _dot(xe, a2[...])
    u = _tv_tanh(u)
    d = _dot(u, k2[...])
    dxe_out[...] = d
    xe_out[...] = xe - H * d


def _edge1(xe, g1, g2, a2t, ke2t):
    e = xe.shape[0]
    grid = (e // EDGE_BLOCK,)
    eb = pl.BlockSpec((EDGE_BLOCK, 64), lambda i: (i, 0))
    shp = jax.ShapeDtypeStruct((e, 64), jnp.float32)
    return pl.pallas_call(
        _edge1_body,
        grid=grid,
        in_specs=[eb, eb, eb, _w_spec(), _w_spec()],
        out_specs=[eb, eb],
        out_shape=(shp, shp),
    )(xe, g1, g2, a2t, ke2t)


def _node_pass_body(p_ref, xn_ref, d1, d2, c2, k2, b1n, b2n,
                    xn_out, z1_out, z2_out):
    n = xn_ref.shape[0]
    s1 = p_ref[0:n, :] + p_ref[2 * n:3 * n, :]
    s2 = p_ref[n:2 * n, :] + p_ref[3 * n:4 * n, :]
    t = _dot(s1, d1[...]) + _dot(s2, d2[...]) + _dot(xn_ref[...], c2[...])
    t = _tv_tanh(t)
    xn = xn_ref[...] - H * _dot(t, k2[...])
    xn_out[...] = xn
    z1_out[...] = _dot(xn, b1n[...])
    z2_out[...] = _dot(xn, b2n[...])


def _node_pass(partials, xn, d1, d2, c2t, kn2t, b1n, b2n):
    n = xn.shape[0]
    shp = jax.ShapeDtypeStruct((n, 64), jnp.float32)
    return pl.pallas_call(
        _node_pass_body,
        grid=(1,),
        in_specs=[
            pl.BlockSpec((4 * n, 64), lambda i: (0, 0)),
            pl.BlockSpec((n, 64), lambda i: (0, 0)),
            _w_spec(), _w_spec(), _w_spec(), _w_spec(), _w_spec(), _w_spec(),
        ],
        out_specs=[pl.BlockSpec((n, 64), lambda i: (0, 0))] * 3,
        out_shape=(shp, shp, shp),
    )(partials, xn, d1, d2, c2t, kn2t, b1n, b2n)


def _node_last_body(p_ref, xn_ref, d1, d2, c2, k2, kc, out_ref):
    n = xn_ref.shape[0]
    s1 = p_ref[0:n, :] + p_ref[2 * n:3 * n, :]
    s2 = p_ref[n:2 * n, :] + p_ref[3 * n:4 * n, :]
    t = _dot(s1, d1[...]) + _dot(s2, d2[...]) + _dot(xn_ref[...], c2[...])
    t = _tv_tanh(t)
    xn = xn_ref[...] - H * _dot(t, k2[...])
    out_ref[...] = _dot(xn, kc[...])


def _node_last(partials, xn, d1, d2, c2t, kn2t, kct):
    n = xn.shape[0]
    return pl.pallas_call(
        _node_last_body,
        grid=(1,),
        in_specs=[
            pl.BlockSpec((4 * n, 64), lambda i: (0, 0)),
            pl.BlockSpec((n, 64), lambda i: (0, 0)),
            _w_spec(), _w_spec(), _w_spec(), _w_spec(), _w_spec(),
        ],
        out_specs=pl.BlockSpec((n, 64), lambda i: (0, 0)),
        out_shape=jax.ShapeDtypeStruct((n, 64), jnp.float32),
    )(partials, xn, d1, d2, c2t, kn2t, kct)


def kernel(xn, xe, edge_index, K1Nopen, K2Nopen, K1Eopen, K2Eopen, KNclose,
           KE1, KE2, KN1, KN2):
    n = xn.shape[2]
    nlayers = KE1.shape[0]
    i_ind = edge_index[0]
    j_ind = edge_index[1]

    ew = []
    nwts = []
    for i in range(nlayers):
        a1t = KE1[i, :, 0:64].T
        a2t = KE1[i, :, 64:128].T
        a3t = KE1[i, :, 128:192].T
        ew.append((0.5 * a1t + a3t, 0.5 * a1t - a3t, a2t, KE2[i].T))
        c1t = KN1[i, :, 0:64].T
        c2t = KN1[i, :, 64:128].T
        c3t = KN1[i, :, 128:192].T
        nwts.append((0.5 * c1t + c3t, 0.5 * c1t - c3t, c2t, KN2[i].T))

    xn0 = xn[0].T
    xe16 = xe[0].T
    zeros_n = jnp.zeros((n, 64), jnp.float32)

    xn_r, z1, z2 = _node_open(xn0, K1Nopen.T, K2Nopen.T, ew[0][0], ew[0][1])

    xe_r = None
    xn_c = None
    for i in range(nlayers):
        b1, b2, a2t, ke2t = ew[i]
        d1, d2, c2t, kn2t = nwts[i]
        g1, g2 = _sc_gather(z1, z2, i_ind, j_ind)
        if i == 0:
            xe_r, dxe = _edge0(xe16, g1, g2, K1Eopen.T, K2Eopen.T, a2t, ke2t)
        else:
            xe_r, dxe = _edge1(xe_r, g1, g2, a2t, ke2t)
        partials = _sc_scatter(dxe, i_ind, j_ind, zeros_n)
        partials = partials.reshape(NC * 2 * n, 64)
        if i + 1 < nlayers:
            nb1, nb2 = ew[i + 1][0], ew[i + 1][1]
            xn_r, z1, z2 = _node_pass(partials, xn_r, d1, d2, c2t, kn2t,
                                      nb1, nb2)
        else:
            xn_c = _node_last(partials, xn_r, d1, d2, c2t, kn2t, KNclose.T)

    out_xn = xn_c.T[None]
    out_xe = xe_r.T[None]
    return (out_xn, out_xe)


# speedup vs baseline: 3.6388x; 3.6388x over previous
import functools

import jax
import jax.numpy as jnp
from jax import lax
from jax.experimental import pallas as pl
from jax.experimental.pallas import tpu as pltpu
from jax.experimental.pallas import tpu_sc as plsc

EPS = 0.001
H = 0.1
NC, NS = 2, 16
NW = NC * NS
CH = 400
EDGE_BLOCK = 4000


def _tv_tanh(t):
    t = t - jnp.mean(t, axis=1, keepdims=True)
    t = t * lax.rsqrt(jnp.sum(t * t, axis=1, keepdims=True) + EPS)
    return jnp.tanh(t)


def _dot(a, b):
    return jnp.dot(a, b, preferred_element_type=jnp.float32)


def _sc_mesh():
    return plsc.VectorSubcoreMesh(
        core_axis_name="c", subcore_axis_name="s", num_cores=NC,
        num_subcores=NS)


_SC_PARAMS = pltpu.CompilerParams(use_tc_tiling_on_sc=False)


def _sc_gather(z1, z2, i_ind, j_ind):
    e = i_ind.shape[0]
    per = e // NW
    steps = per // CH

    @functools.partial(
        pl.kernel,
        out_type=(jax.ShapeDtypeStruct((e, 64), jnp.float32),
                  jax.ShapeDtypeStruct((e, 64), jnp.float32)),
        mesh=_sc_mesh(),
        scratch_types=[
            pltpu.VMEM((CH,), jnp.int32),
            pltpu.VMEM((CH,), jnp.int32),
            pltpu.VMEM((CH, 64), jnp.float32),
            pltpu.VMEM((CH, 64), jnp.float32),
            pltpu.SemaphoreType.DMA,
            pltpu.SemaphoreType.DMA,
        ],
        compiler_params=_SC_PARAMS,
    )
    def k(z1_hbm, z2_hbm, i_hbm, j_hbm, g1_hbm, g2_hbm,
          idx1_v, idx2_v, buf1_v, buf2_v, sem1, sem2):
        wid = lax.axis_index("s") * NC + lax.axis_index("c")
        base = wid * per

        def body(t, carry):
            start = base + t * CH
            pltpu.sync_copy(i_hbm.at[pl.ds(start, CH)], idx1_v)
            pltpu.sync_copy(j_hbm.at[pl.ds(start, CH)], idx2_v)
            d1 = pltpu.async_copy(z1_hbm.at[idx1_v], buf1_v, sem1)
            d2 = pltpu.async_copy(z2_hbm.at[idx2_v], buf2_v, sem2)
            d1.wait()
            d2.wait()
            pltpu.sync_copy(buf1_v, g1_hbm.at[pl.ds(start, CH)])
            pltpu.sync_copy(buf2_v, g2_hbm.at[pl.ds(start, CH)])
            return carry

        lax.fori_loop(0, steps, body, 0)

    return k(z1, z2, i_ind, j_ind)


def _sc_scatter(flux, i_ind, j_ind, zeros_n):
    e = i_ind.shape[0]
    n = zeros_n.shape[0]
    per = e // NW
    steps = per // CH
    stripe = n // NS

    @functools.partial(
        pl.kernel,
        out_type=jax.ShapeDtypeStruct((NC, 2 * n, 64), jnp.float32),
        mesh=_sc_mesh(),
        scratch_types=[
            pltpu.VMEM((CH,), jnp.int32),
            pltpu.VMEM((CH,), jnp.int32),
            pltpu.VMEM((CH, 64), jnp.float32),
            pltpu.VMEM_SHARED((n, 64), jnp.float32),
            pltpu.VMEM_SHARED((n, 64), jnp.float32),
        ],
        compiler_params=_SC_PARAMS,
    )
    def k(flux_hbm, i_hbm, j_hbm, z_hbm, out_hbm,
          idx1_v, idx2_v, buf_v, acc1, acc2):
        cid = lax.axis_index("c")
        sid = lax.axis_index("s")
        wid = sid * NC + cid
        row0 = sid * stripe
        pltpu.sync_copy(z_hbm.at[pl.ds(row0, stripe)],
                        acc1.at[pl.ds(row0, stripe)])
        pltpu.sync_copy(z_hbm.at[pl.ds(row0, stripe)],
                        acc2.at[pl.ds(row0, stripe)])
        plsc.subcore_barrier()

        base = wid * per

        def body(t, carry):
            start = base + t * CH
            pltpu.sync_copy(i_hbm.at[pl.ds(start, CH)], idx1_v)
            pltpu.sync_copy(j_hbm.at[pl.ds(start, CH)], idx2_v)
            pltpu.sync_copy(flux_hbm.at[pl.ds(start, CH)], buf_v)
            pltpu.sync_copy(buf_v, acc1.at[idx1_v], add=True)
            pltpu.sync_copy(buf_v, acc2.at[idx2_v], add=True)
            return carry

        lax.fori_loop(0, steps, body, 0)
        plsc.subcore_barrier()
        pltpu.sync_copy(acc1.at[pl.ds(row0, stripe)],
                        out_hbm.at[cid, pl.ds(row0, stripe)])
        pltpu.sync_copy(acc2.at[pl.ds(row0, stripe)],
                        out_hbm.at[cid, pl.ds(n + row0, stripe)])

    return k(flux, i_ind, j_ind, zeros_n)


def _w_spec():
    return pl.BlockSpec((64, 64), lambda i: (0, 0))


def _node_open_body(x_ref, w1, w2, b1, b2, xn_out, z1_out, z2_out):
    t = _dot(x_ref[...], w1[...])
    t = _tv_tanh(t)
    xn = _dot(t, w2[...])
    xn_out[...] = xn
    z1_out[...] = _dot(xn, b1[...])
    z2_out[...] = _dot(xn, b2[...])


def _node_open(xn0, w1t, w2t, b1, b2):
    n = xn0.shape[0]
    shp = jax.ShapeDtypeStruct((n, 64), jnp.float32)
    return pl.pallas_call(
        _node_open_body,
        grid=(1,),
        in_specs=[
            pl.BlockSpec((n, 128), lambda i: (0, 0)),
            pl.BlockSpec((128, 64), lambda i: (0, 0)),
            _w_spec(), _w_spec(), _w_spec(),
        ],
        out_specs=[pl.BlockSpec((n, 64), lambda i: (0, 0))] * 3,
        out_shape=(shp, shp, shp),
    )(xn0, w1t, w2t, b1, b2)


def _edge0_body(xe_ref, g1_ref, g2_ref, w1, w2, a2, k2, xe_out, dxe_out):
    t = _dot(xe_ref[...], w1[...])
    t = _tv_tanh(t)
    xe = _dot(t, w2[...])
    u = g1_ref[...] + g2_ref[...] + _dot(xe, a2[...])
    u = _tv_tanh(u)
    d = _dot(u, k2[...])
    dxe_out[...] = d
    xe_out[...] = xe - H * d


def _edge0(xe16, g1, g2, w1t, w2t, a2t, ke2t):
    e = xe16.shape[0]
    grid = (e // EDGE_BLOCK,)
    eb = pl.BlockSpec((EDGE_BLOCK, 64), lambda i: (i, 0))
    shp = jax.ShapeDtypeStruct((e, 64), jnp.float32)
    return pl.pallas_call(
        _edge0_body,
        grid=grid,
        in_specs=[
            pl.BlockSpec((EDGE_BLOCK, 16), lambda i: (i, 0)),
            eb, eb,
            pl.BlockSpec((16, 64), lambda i: (0, 0)),
            _w_spec(), _w_spec(), _w_spec(),
        ],
        out_specs=[eb, eb],
        out_shape=(shp, shp),
    )(xe16, g1, g2, w1t, w2t, a2t, ke2t)


def _edge1_body(xe_ref, g1_ref, g2_ref, a2, k2, xe_out, dxe_out):
    xe = xe_ref[...]
    u = g1_ref[...] + g2_ref[...] + _dot(xe, a2[...])
    u = _tv_tanh(u)
    d = _dot(u, k2[...])
    dxe_out[...] = d
    xe_out[...] = xe - H * d


def _edge1(xe, g1, g2, a2t, ke2t):
    e = xe.shape[0]
    grid = (e // EDGE_BLOCK,)
    eb = pl.BlockSpec((EDGE_BLOCK, 64), lambda i: (i, 0))
    shp = jax.ShapeDtypeStruct((e, 64), jnp.float32)
    return pl.pallas_call(
        _edge1_body,
        grid=grid,
        in_specs=[eb, eb, eb, _w_spec(), _w_spec()],
        out_specs=[eb, eb],
        out_shape=(shp, shp),
    )(xe, g1, g2, a2t, ke2t)


def _node_pass_body(p_ref, xn_ref, d1, d2, c2, k2, b1n, b2n,
                    xn_out, z1_out, z2_out):
    n = xn_ref.shape[0]
    s1 = p_ref[0:n, :] + p_ref[2 * n:3 * n, :]
    s2 = p_ref[n:2 * n, :] + p_ref[3 * n:4 * n, :]
    t = _dot(s1, d1[...]) + _dot(s2, d2[...]) + _dot(xn_ref[...], c2[...])
    t = _tv_tanh(t)
    xn = xn_ref[...] - H * _dot(t, k2[...])
    xn_out[...] = xn
    z1_out[...] = _dot(xn, b1n[...])
    z2_out[...] = _dot(xn, b2n[...])


def _node_pass(partials, xn, d1, d2, c2t, kn2t, b1n, b2n):
    n = xn.shape[0]
    shp = jax.ShapeDtypeStruct((n, 64), jnp.float32)
    return pl.pallas_call(
        _node_pass_body,
        grid=(1,),
        in_specs=[
            pl.BlockSpec((4 * n, 64), lambda i: (0, 0)),
            pl.BlockSpec((n, 64), lambda i: (0, 0)),
            _w_spec(), _w_spec(), _w_spec(), _w_spec(), _w_spec(), _w_spec(),
        ],
        out_specs=[pl.BlockSpec((n, 64), lambda i: (0, 0))] * 3,
        out_shape=(shp, shp, shp),
    )(partials, xn, d1, d2, c2t, kn2t, b1n, b2n)


def _node_last_body(p_ref, xn_ref, d1, d2, c2, k2, kc, out_ref):
    n = xn_ref.shape[0]
    s1 = p_ref[0:n, :] + p_ref[2 * n:3 * n, :]
    s2 = p_ref[n:2 * n, :] + p_ref[3 * n:4 * n, :]
    t = _dot(s1, d1[...]) + _dot(s2, d2[...]) + _dot(xn_ref[...], c2[...])
    t = _tv_tanh(t)
    xn = xn_ref[...] - H * _dot(t, k2[...])
    out_ref[...] = _dot(xn, kc[...])


def _node_last(partials, xn, d1, d2, c2t, kn2t, kct):
    n = xn.shape[0]
    return pl.pallas_call(
        _node_last_body,
        grid=(1,),
        in_specs=[
            pl.BlockSpec((4 * n, 64), lambda i: (0, 0)),
            pl.BlockSpec((n, 64), lambda i: (0, 0)),
            _w_spec(), _w_spec(), _w_spec(), _w_spec(), _w_spec(),
        ],
        out_specs=pl.BlockSpec((n, 64), lambda i: (0, 0)),
        out_shape=jax.ShapeDtypeStruct((n, 64), jnp.float32),
    )(partials, xn, d1, d2, c2t, kn2t, kct)


def kernel(xn, xe, edge_index, K1Nopen, K2Nopen, K1Eopen, K2Eopen, KNclose,
           KE1, KE2, KN1, KN2):
    n = xn.shape[2]
    nlayers = KE1.shape[0]
    i_ind = edge_index[0]
    j_ind = edge_index[1]

    ew = []
    nwts = []
    for i in range(nlayers):
        a1t = KE1[i, :, 0:64].T
        a2t = KE1[i, :, 64:128].T
        a3t = KE1[i, :, 128:192].T
        ew.append((0.5 * a1t + a3t, 0.5 * a1t - a3t, a2t, KE2[i].T))
        c1t = KN1[i, :, 0:64].T
        c2t = KN1[i, :, 64:128].T
        c3t = KN1[i, :, 128:192].T
        nwts.append((0.5 * c1t + c3t, 0.5 * c1t - c3t, c2t, KN2[i].T))

    xn0 = xn[0].T
    xe16 = xe[0].T
    zeros_n = jnp.zeros((n, 64), jnp.float32)

    xn_r, z1, z2 = _node_open(xn0, K1Nopen.T, K2Nopen.T, ew[0][0], ew[0][1])

    xe_r = None
    xn_c = None
    for i in range(nlayers):
        b1, b2, a2t, ke2t = ew[i]
        d1, d2, c2t, kn2t = nwts[i]
        g1, g2 = _sc_gather(z1, z2, i_ind, j_ind)
        if i == 0:
            xe_r, dxe = _edge0(xe16, g1, g2, K1Eopen.T, K2Eopen.T, a2t, ke2t)
        else:
            xe_r, dxe = _edge1(xe_r, g1, g2, a2t, ke2t)
        partials = _sc_scatter(dxe, i_ind, j_ind, zeros_n)
        partials = partials.reshape(NC * 2 * n, 64)
        if i + 1 < nlayers:
            nb1, nb2 = ew[i + 1][0], ew[i + 1][1]
            xn_r, z1, z2 = _node_pass(partials, xn_r, d1, d2, c2t, kn2t,
                                      nb1, nb2)
        else:
            xn_c = _node_last(partials, xn_r, d1, d2, c2t, kn2t, KNclose.T)

    out_xn = xn_c.T[None]
    out_xe = xe_r.T[None]
    return (out_xn, out_xe)


# speedup vs baseline: 3.8246x; 1.0511x over previous
import functools

import jax
import jax.numpy as jnp
from jax import lax
from jax.experimental import pallas as pl
from jax.experimental.pallas import tpu as pltpu
from jax.experimental.pallas import tpu_sc as plsc

EPS = 0.001
H = 0.1
NC, NS = 2, 16
NW = NC * NS
CH = 400
SCH = 800
EDGE_BLOCK = 4000


def _norm_tanh(t, ones):
    s = _dot(t * t, ones)
    return jnp.tanh(t * lax.rsqrt(s + EPS))


def _dot(a, b):
    return jnp.dot(a, b, preferred_element_type=jnp.float32)


_ONES64 = None


def _sc_mesh():
    return plsc.VectorSubcoreMesh(
        core_axis_name="c", subcore_axis_name="s", num_cores=NC,
        num_subcores=NS)


_SC_PARAMS = pltpu.CompilerParams(use_tc_tiling_on_sc=False)


def _sc_gather(z1, z2, i3, j3):
    nwk, steps, ch = i3.shape
    per = steps * ch
    e = NW * per

    @functools.partial(
        pl.kernel,
        out_type=(jax.ShapeDtypeStruct((e, 64), jnp.float32),
                  jax.ShapeDtypeStruct((e, 64), jnp.float32)),
        mesh=_sc_mesh(),
        scratch_types=[
            pltpu.VMEM((steps, ch), jnp.int32),
            pltpu.VMEM((steps, ch), jnp.int32),
            pltpu.VMEM((ch, 64), jnp.float32),
            pltpu.VMEM((ch, 64), jnp.float32),
            pltpu.VMEM((ch, 64), jnp.float32),
            pltpu.VMEM((ch, 64), jnp.float32),
        ] + [pltpu.SemaphoreType.DMA] * 8,
        compiler_params=_SC_PARAMS,
    )
    def k(z1_hbm, z2_hbm, i_hbm, j_hbm, g1_hbm, g2_hbm,
          idx1_v, idx2_v, a0, b0, a1, b1,
          sg1_0, sg2_0, sg1_1, sg2_1, sw1_0, sw2_0, sw1_1, sw2_1):
        wid = lax.axis_index("s") * NC + lax.axis_index("c")
        base = wid * per
        pltpu.sync_copy(i_hbm.at[wid], idx1_v)
        pltpu.sync_copy(j_hbm.at[wid], idx2_v)
        bufs = [(a0, b0), (a1, b1)]
        gsems = [(sg1_0, sg2_0), (sg1_1, sg2_1)]
        wsems = [(sw1_0, sw2_0), (sw1_1, sw2_1)]
        wdesc = [None, None]
        gdesc = [None, None]
        prev = None
        for t in range(steps):
            slot = t % 2
            if wdesc[slot] is not None:
                wdesc[slot][0].wait()
                wdesc[slot][1].wait()
            gdesc[slot] = (
                pltpu.async_copy(z1_hbm.at[idx1_v.at[t]], bufs[slot][0],
                                 gsems[slot][0]),
                pltpu.async_copy(z2_hbm.at[idx2_v.at[t]], bufs[slot][1],
                                 gsems[slot][1]),
            )
            if prev is not None:
                tp, sp = prev
                gdesc[sp][0].wait()
                gdesc[sp][1].wait()
                wdesc[sp] = (
                    pltpu.async_copy(
                        bufs[sp][0], g1_hbm.at[pl.ds(base + tp * ch, ch)],
                        wsems[sp][0]),
                    pltpu.async_copy(
                        bufs[sp][1], g2_hbm.at[pl.ds(base + tp * ch, ch)],
                        wsems[sp][1]),
                )
            prev = (t, slot)
        tl, sl = prev
        gdesc[sl][0].wait()
        gdesc[sl][1].wait()
        w1 = pltpu.async_copy(bufs[sl][0],
                              g1_hbm.at[pl.ds(base + tl * ch, ch)],
                              wsems[sl][0])
        w2 = pltpu.async_copy(bufs[sl][1],
                              g2_hbm.at[pl.ds(base + tl * ch, ch)],
                              wsems[sl][1])
        w1.wait()
        w2.wait()
        if wdesc[1 - sl] is not None:
            wdesc[1 - sl][0].wait()
            wdesc[1 - sl][1].wait()

    return k(z1, z2, i3, j3)


def _sc_scatter(flux, ij_cat, zeros_n):
    e = ij_cat.shape[0] // 2
    per = e // NS
    steps = per // SCH
    n = zeros_n.shape[0]
    stripe = n // NS

    @functools.partial(
        pl.kernel,
        out_type=jax.ShapeDtypeStruct((NC, n, 64), jnp.float32),
        mesh=_sc_mesh(),
        scratch_types=[
            pltpu.VMEM((SCH,), jnp.int32),
            pltpu.VMEM((SCH, 64), jnp.float32),
            pltpu.VMEM((SCH, 64), jnp.float32),
            pltpu.VMEM_SHARED((n, 64), jnp.float32),
            pltpu.SemaphoreType.DMA,
            pltpu.SemaphoreType.DMA,
        ],
        compiler_params=_SC_PARAMS,
    )
    def k(flux_hbm, ij_hbm, z_hbm, out_hbm,
          idx_v, f0, f1, acc, sem0, sem1):
        cid = lax.axis_index("c")
        sid = lax.axis_index("s")
        row0 = sid * stripe
        pltpu.sync_copy(z_hbm.at[pl.ds(row0, stripe)],
                        acc.at[pl.ds(row0, stripe)])
        plsc.subcore_barrier()

        base = sid * per
        idx_base = cid * e + base

        def body(t, carry):
            start = base + t * SCH
            d1 = pltpu.async_copy(
                ij_hbm.at[pl.ds(idx_base + t * SCH, SCH)], idx_v, sem1)
            d0 = pltpu.async_copy(
                flux_hbm.at[pl.ds(start, SCH)], f0, sem0)
            d1.wait()
            d0.wait()
            pltpu.sync_copy(f0, acc.at[idx_v], add=True)
            return carry

        lax.fori_loop(0, steps, body, 0)
        plsc.subcore_barrier()
        pltpu.sync_copy(acc.at[pl.ds(row0, stripe)],
                        out_hbm.at[cid, pl.ds(row0, stripe)])

    return k(flux, ij_cat, zeros_n)


def _w_spec():
    return pl.BlockSpec((64, 64), lambda i: (0, 0))


def _node_open_body(x_ref, w1, w2, b1, b2, ones, xn_out, z1_out, z2_out):
    t = _dot(x_ref[...], w1[...])
    t = _norm_tanh(t, ones[...])
    xn = _dot(t, w2[...])
    xn_out[...] = xn
    z1_out[...] = _dot(xn, b1[...])
    z2_out[...] = _dot(xn, b2[...])


def _node_open(xn0, w1t, w2t, b1, b2):
    n = xn0.shape[0]
    shp = jax.ShapeDtypeStruct((n, 64), jnp.float32)
    return pl.pallas_call(
        _node_open_body,
        grid=(1,),
        in_specs=[
            pl.BlockSpec((n, 128), lambda i: (0, 0)),
            pl.BlockSpec((128, 64), lambda i: (0, 0)),
            _w_spec(), _w_spec(), _w_spec(), _w_spec(),
        ],
        out_specs=[pl.BlockSpec((n, 64), lambda i: (0, 0))] * 3,
        out_shape=(shp, shp, shp),
    )(xn0, w1t, w2t, b1, b2, _ONES64)


def _edge0_body(xe_ref, g1_ref, g2_ref, w1, w2, a2, k2, ones, xe_out, dxe_out):
    t = _dot(xe_ref[...], w1[...])
    t = _norm_tanh(t, ones[...])
    xe = _dot(t, w2[...])
    u = g1_ref[...] + g2_ref[...] + _dot(xe, a2[...])
    u = _norm_tanh(u, ones[...])
    d = _dot(u, k2[...])
    dxe_out[...] = d
    xe_out[...] = xe - H * d


def _edge0(xe16, g1, g2, w1t, w2t, a2t, ke2t):
    e = xe16.shape[0]
    grid = (e // EDGE_BLOCK,)
    eb = pl.BlockSpec((EDGE_BLOCK, 64), lambda i: (i, 0))
    shp = jax.ShapeDtypeStruct((e, 64), jnp.float32)
    return pl.pallas_call(
        _edge0_body,
        grid=grid,
        in_specs=[
            pl.BlockSpec((EDGE_BLOCK, 16), lambda i: (i, 0)),
            eb, eb,
            pl.BlockSpec((16, 64), lambda i: (0, 0)),
            _w_spec(), _w_spec(), _w_spec(), _w_spec(),
        ],
        out_specs=[eb, eb],
        out_shape=(shp, shp),
    )(xe16, g1, g2, w1t, w2t, a2t, ke2t, _ONES64)


def _edge1_body(xe_ref, g1_ref, g2_ref, a2, k2, ones, xe_out, dxe_out):
    xe = xe_ref[...]
    u = g1_ref[...] + g2_ref[...] + _dot(xe, a2[...])
    u = _norm_tanh(u, ones[...])
    d = _dot(u, k2[...])
    dxe_out[...] = d
    xe_out[...] = xe - H * d


def _edge1(xe, g1, g2, a2t, ke2t):
    e = xe.shape[0]
    grid = (e // EDGE_BLOCK,)
    eb = pl.BlockSpec((EDGE_BLOCK, 64), lambda i: (i, 0))
    shp = jax.ShapeDtypeStruct((e, 64), jnp.float32)
    return pl.pallas_call(
        _edge1_body,
        grid=grid,
        in_specs=[eb, eb, eb, _w_spec(), _w_spec(), _w_spec()],
        out_specs=[eb, eb],
        out_shape=(shp, shp),
    )(xe, g1, g2, a2t, ke2t, _ONES64)


def _node_pass_body(p_ref, xn_ref, d1, d2, c2, k2, b1n, b2n, ones,
                    xn_out, z1_out, z2_out):
    n = xn_ref.shape[0]
    s1 = p_ref[0:n, :]
    s2 = p_ref[n:2 * n, :]
    t = _dot(s1, d1[...]) + _dot(s2, d2[...]) + _dot(xn_ref[...], c2[...])
    t = _norm_tanh(t, ones[...])
    xn = xn_ref[...] - H * _dot(t, k2[...])
    xn_out[...] = xn
    z1_out[...] = _dot(xn, b1n[...])
    z2_out[...] = _dot(xn, b2n[...])


def _node_pass(partials, xn, d1, d2, c2t, kn2t, b1n, b2n):
    n = xn.shape[0]
    shp = jax.ShapeDtypeStruct((n, 64), jnp.float32)
    return pl.pallas_call(
        _node_pass_body,
        grid=(1,),
        in_specs=[
            pl.BlockSpec((2 * n, 64), lambda i: (0, 0)),
            pl.BlockSpec((n, 64), lambda i: (0, 0)),
            _w_spec(), _w_spec(), _w_spec(), _w_spec(), _w_spec(), _w_spec(),
            _w_spec(),
        ],
        out_specs=[pl.BlockSpec((n, 64), lambda i: (0, 0))] * 3,
        out_shape=(shp, shp, shp),
    )(partials, xn, d1, d2, c2t, kn2t, b1n, b2n, _ONES64)


def _node_last_body(p_ref, xn_ref, d1, d2, c2, k2, kc, ones, out_ref):
    n = xn_ref.shape[0]
    s1 = p_ref[0:n, :]
    s2 = p_ref[n:2 * n, :]
    t = _dot(s1, d1[...]) + _dot(s2, d2[...]) + _dot(xn_ref[...], c2[...])
    t = _norm_tanh(t, ones[...])
    xn = xn_ref[...] - H * _dot(t, k2[...])
    out_ref[...] = _dot(xn, kc[...])


def _node_last(partials, xn, d1, d2, c2t, kn2t, kct):
    n = xn.shape[0]
    return pl.pallas_call(
        _node_last_body,
        grid=(1,),
        in_specs=[
            pl.BlockSpec((2 * n, 64), lambda i: (0, 0)),
            pl.BlockSpec((n, 64), lambda i: (0, 0)),
            _w_spec(), _w_spec(), _w_spec(), _w_spec(), _w_spec(), _w_spec(),
        ],
        out_specs=pl.BlockSpec((n, 64), lambda i: (0, 0)),
        out_shape=jax.ShapeDtypeStruct((n, 64), jnp.float32),
    )(partials, xn, d1, d2, c2t, kn2t, kct, _ONES64)


def kernel(xn, xe, edge_index, K1Nopen, K2Nopen, K1Eopen, K2Eopen, KNclose,
           KE1, KE2, KN1, KN2):
    n = xn.shape[2]
    nlayers = KE1.shape[0]
    i_ind = edge_index[0]
    j_ind = edge_index[1]

    cm = jnp.eye(64, dtype=jnp.float32) - 1.0 / 64.0
    ew = []
    nwts = []
    for i in range(nlayers):
        a1t = KE1[i, :, 0:64].T
        a2t = KE1[i, :, 64:128].T
        a3t = KE1[i, :, 128:192].T
        ew.append(((0.5 * a1t + a3t) @ cm, (0.5 * a1t - a3t) @ cm,
                   a2t @ cm, KE2[i].T))
        c1t = KN1[i, :, 0:64].T
        c2t = KN1[i, :, 64:128].T
        c3t = KN1[i, :, 128:192].T
        nwts.append(((0.5 * c1t + c3t) @ cm, (0.5 * c1t - c3t) @ cm,
                     c2t @ cm, KN2[i].T))

    xn0 = xn[0].T
    xe16 = xe[0].T
    e = xe.shape[2]
    steps = e // (NW * CH)
    i3 = i_ind.reshape(NW, steps, CH)
    j3 = j_ind.reshape(NW, steps, CH)
    zeros_n = jnp.zeros((n, 64), jnp.float32)
    ij_cat = jnp.concatenate([i_ind, j_ind])

    global _ONES64
    _ONES64 = jnp.ones((64, 64), jnp.float32)

    xn_r, z1, z2 = _node_open(xn0, K1Nopen.T @ cm, K2Nopen.T,
                              ew[0][0], ew[0][1])

    xe_r = None
    xn_c = None
    for i in range(nlayers):
        b1, b2, a2t, ke2t = ew[i]
        d1, d2, c2t, kn2t = nwts[i]
        g1, g2 = _sc_gather(z1, z2, i3, j3)
        if i == 0:
            xe_r, dxe = _edge0(xe16, g1, g2, K1Eopen.T @ cm, K2Eopen.T,
                               a2t, ke2t)
        else:
            xe_r, dxe = _edge1(xe_r, g1, g2, a2t, ke2t)
        partials = _sc_scatter(dxe, ij_cat, zeros_n)
        partials = partials.reshape(NC * n, 64)
        if i + 1 < nlayers:
            nb1, nb2 = ew[i + 1][0], ew[i + 1][1]
            xn_r, z1, z2 = _node_pass(partials, xn_r, d1, d2, c2t, kn2t,
                                      nb1, nb2)
        else:
            xn_c = _node_last(partials, xn_r, d1, d2, c2t, kn2t, KNclose.T)

    out_xn = xn_c.T[None]
    out_xe = xe_r.T[None]
    return (out_xn, out_xe)
